# baseline (speedup 1.0000x reference)
import functools

import jax
import jax.numpy as jnp
from jax import lax
from jax.experimental import pallas as pl
from jax.experimental.pallas import tpu as pltpu
from jax.experimental.pallas import tpu_sc as plsc

N = 10000
E = 320000
D = 128
H = 128
C = 2
G = 64
EPS = 1e-5

NP = 10240
NC = 2
NS = 16
L = 16
NW = NC * NS
K = 128
CH = 80
EW = CH * K
EP = NW * EW
SLAB = NP // NS
HIGHEST = jax.lax.Precision.HIGHEST

@functools.cache
def _mesh():
    return plsc.VectorSubcoreMesh(
        core_axis_name="c", subcore_axis_name="s",
        num_cores=NC, num_subcores=NS)


def _sc_deg_body(dst_hbm, out_hbm, didx_v, ones_v, zcol_v, acc_sh):
    core = lax.axis_index("c")
    sub = lax.axis_index("s")
    w = sub * NC + core

    for i in range(K // L):
        ones_v[pl.ds(i * L, L)] = jnp.ones((L,), jnp.float32)

    def _zero(i, _):
        zcol_v[pl.ds(i * L, L)] = jnp.zeros((L,), jnp.float32)
        return 0
    lax.fori_loop(0, SLAB // L, _zero, 0)
    pltpu.sync_copy(zcol_v, acc_sh.at[pl.ds(sub * SLAB, SLAB)])
    plsc.subcore_barrier()

    pltpu.sync_copy(dst_hbm.at[w, pl.ds(0, CH)], didx_v)

    def _chunk(j, _):
        pltpu.sync_copy(ones_v, acc_sh.at[didx_v.at[j]], add=True)
        return 0
    lax.fori_loop(0, CH, _chunk, 0)
    plsc.subcore_barrier()
    pltpu.sync_copy(acc_sh.at[pl.ds(sub * SLAB, SLAB)],
                    out_hbm.at[core, pl.ds(sub * SLAB, SLAB)])


@functools.cache
def _sc_deg():
    return pl.kernel(
        _sc_deg_body,
        out_type=jax.ShapeDtypeStruct((NC, NP), jnp.float32),
        mesh=_mesh(),
        scratch_types=[
            pltpu.VMEM((CH, K), jnp.int32),
            pltpu.VMEM((K,), jnp.float32),
            pltpu.VMEM((SLAB,), jnp.float32),
            pltpu.VMEM_SHARED((NP,), jnp.float32),
        ],
    )


def _sc_agg_body(y_hbm, src_hbm, dst_hbm, out_hbm, sidx0_v, sidx1_v, didx0_v,
                 didx1_v, rows0_v, rows1_v, isem0, isem1, gsem0, gsem1,
                 acc_sh):
    core = lax.axis_index("c")
    sub = lax.axis_index("s")
    w = sub * NC + core
    sidx = (sidx0_v, sidx1_v)
    didx = (didx0_v, didx1_v)
    rows = (rows0_v, rows1_v)
    isem = (isem0, isem1)
    gsem = (gsem0, gsem1)

    def _zero(i, _):
        rows0_v[i // 8, pl.ds((i % 8) * L, L)] = jnp.zeros((L,), jnp.float32)
        return 0
    lax.fori_loop(0, K * (H // L), _zero, 0)
    for t in range(SLAB // K):
        pltpu.sync_copy(rows0_v, acc_sh.at[pl.ds(sub * SLAB + t * K, K)])
    plsc.subcore_barrier()

    def _iload(j, b):
        pltpu.async_copy(src_hbm.at[w, j], sidx[b], isem[b])
        pltpu.async_copy(dst_hbm.at[w, j], didx[b], isem[b])

    def _iwait(j, b):
        pltpu.make_async_copy(src_hbm.at[w, j], sidx[b], isem[b]).wait()
        pltpu.make_async_copy(dst_hbm.at[w, j], didx[b], isem[b]).wait()

    _iload(0, 0)
    _iload(1, 1)
    _iwait(0, 0)
    pltpu.async_copy(y_hbm.at[sidx0_v], rows0_v, gsem0)

    def _pair(i, _):
        for b in range(2):
            j = 2 * i + b
            nb = 1 - b
            _iwait(j + 1, nb)
            pltpu.async_copy(y_hbm.at[sidx[nb]], rows[nb], gsem[nb])
            pltpu.make_async_copy(y_hbm.at[sidx[b]], rows[b], gsem[b]).wait()
            pltpu.sync_copy(rows[b], acc_sh.at[didx[b]], add=True)
            _iload(j + 2, b)
        return 0
    lax.fori_loop(0, CH // 2, _pair, 0)
    pltpu.make_async_copy(y_hbm.at[sidx0_v], rows0_v, gsem0).wait()
    _iwait(CH + 1, 1)
    plsc.subcore_barrier()
    pltpu.sync_copy(acc_sh.at[pl.ds(sub * SLAB, SLAB)],
                    out_hbm.at[core, pl.ds(sub * SLAB, SLAB)])


@functools.cache
def _sc_agg():
    return pl.kernel(
        _sc_agg_body,
        out_type=jax.ShapeDtypeStruct((NC, NP, H), jnp.float32),
        mesh=_mesh(),
        scratch_types=[
            pltpu.VMEM((K,), jnp.int32),
            pltpu.VMEM((K,), jnp.int32),
            pltpu.VMEM((K,), jnp.int32),
            pltpu.VMEM((K,), jnp.int32),
            pltpu.VMEM((K, H), jnp.float32),
            pltpu.VMEM((K, H), jnp.float32),
            pltpu.SemaphoreType.DMA,
            pltpu.SemaphoreType.DMA,
            pltpu.SemaphoreType.DMA,
            pltpu.SemaphoreType.DMA,
            pltpu.VMEM_SHARED((NP, H), jnp.float32),
        ],
    )


def _tc_pre_body(x_ref, w1_ref, d0_ref, d1_ref, y1_ref, dinv_ref):
    deg = d0_ref[...] + d1_ref[...] + 1.0
    dinv = 1.0 / jnp.sqrt(deg)
    xw = jnp.dot(x_ref[...], w1_ref[...], precision=HIGHEST)
    y1_ref[...] = dinv * xw
    dinv_ref[...] = dinv


def _tc_stats_body(a0_ref, a1_ref, y_ref, dinv_ref, b_ref, batr_ref,
                   h_ref, scale_ref, shift_ref):
    h = dinv_ref[...] * (a0_ref[...] + a1_ref[...] + y_ref[...]) + b_ref[...]
    oht = (batr_ref[...] ==
           lax.broadcasted_iota(jnp.int32, (G, NP), 0)).astype(jnp.float32)
    rc = 1.0 / jnp.maximum(jnp.sum(oht, axis=1, keepdims=True), 1.0)
    mean = jnp.dot(oht, h, precision=HIGHEST) * rc
    ex2 = jnp.dot(oht, h * h, precision=HIGHEST) * rc
    var = ex2 - mean * mean
    scale = 1.0 / jnp.sqrt(var + EPS)
    h_ref[...] = h
    scale_ref[...] = scale
    shift_ref[...] = -mean * scale


def _tc_apply_body(h_ref, scale_ref, shift_ref, batc_ref, dinv_ref, w2_ref,
                   y2_ref):
    oh = (batc_ref[...] ==
          lax.broadcasted_iota(jnp.int32, (NP, G), 1)).astype(jnp.float32)
    hn = (h_ref[...] * jnp.dot(oh, scale_ref[...], precision=HIGHEST)
          + jnp.dot(oh, shift_ref[...], precision=HIGHEST))
    h1 = jnp.maximum(hn, 0.0)
    y2_ref[...] = dinv_ref[...] * jnp.dot(h1, w2_ref[...], precision=HIGHEST)


def _tc_pool_body(h_ref, scale_ref, shift_ref, batc_ref, batr_ref, wfc_ref,
                  bfc_ref, out_ref):
    oh = (batc_ref[...] ==
          lax.broadcasted_iota(jnp.int32, (NP, G), 1)).astype(jnp.float32)
    oht = (batr_ref[...] ==
           lax.broadcasted_iota(jnp.int32, (G, NP), 0)).astype(jnp.float32)
    rc = 1.0 / jnp.maximum(jnp.sum(oht, axis=1, keepdims=True), 1.0)
    hn = (h_ref[...] * jnp.dot(oh, scale_ref[...], precision=HIGHEST)
          + jnp.dot(oh, shift_ref[...], precision=HIGHEST))
    h2 = jnp.maximum(hn, 0.0)
    pooled = jnp.dot(oht, h2, precision=HIGHEST) * rc
    out_ref[...] = jnp.dot(pooled, wfc_ref[...], precision=HIGHEST) + bfc_ref[...]


_tc_pre = pl.pallas_call(
    _tc_pre_body,
    out_shape=[jax.ShapeDtypeStruct((NP, H), jnp.float32),
               jax.ShapeDtypeStruct((NP, 1), jnp.float32)])

_tc_stats = pl.pallas_call(
    _tc_stats_body,
    out_shape=[jax.ShapeDtypeStruct((NP, H), jnp.float32),
               jax.ShapeDtypeStruct((G, H), jnp.float32),
               jax.ShapeDtypeStruct((G, H), jnp.float32)])

_tc_apply = pl.pallas_call(
    _tc_apply_body,
    out_shape=jax.ShapeDtypeStruct((NP, H), jnp.float32))

_tc_pool = pl.pallas_call(
    _tc_pool_body,
    out_shape=jax.ShapeDtypeStruct((G, 128), jnp.float32))


def kernel(x, edge_index, batch, W1, b1, W2, b2, Wfc, bfc):
    ei = jnp.pad(edge_index, ((0, 0), (0, EP - E)),
                 constant_values=NP - 1).reshape(2, NW, CH, K)
    src3 = jnp.pad(ei[0], ((0, 0), (0, 2), (0, 0)), constant_values=NP - 1)
    dst3 = jnp.pad(ei[1], ((0, 0), (0, 2), (0, 0)), constant_values=NP - 1)
    x_p = jnp.pad(x, ((0, NP - N), (0, 0)))
    batc = jnp.pad(batch.astype(jnp.int32), (0, NP - N),
                   constant_values=G).reshape(NP, 1)
    batr = batc.reshape(1, NP)
    wfc_p = jnp.pad(Wfc, ((0, 0), (0, 128 - C)))
    bfc_p = jnp.pad(bfc, (0, 128 - C)).reshape(1, 128)

    degp = _sc_deg()(dst3)
    d0 = degp[0].reshape(NP, 1)
    d1 = degp[1].reshape(NP, 1)
    y1, dinv = _tc_pre(x_p, W1, d0, d1)

    ag1 = _sc_agg()(y1, src3, dst3)
    h1, sc1, sh1 = _tc_stats(ag1[0], ag1[1], y1, dinv, b1.reshape(1, H), batr)
    y2 = _tc_apply(h1, sc1, sh1, batc, dinv, W2)

    ag2 = _sc_agg()(y2, src3, dst3)
    h2, sc2, sh2 = _tc_stats(ag2[0], ag2[1], y2, dinv, b2.reshape(1, H), batr)
    outp = _tc_pool(h2, sc2, sh2, batc, batr, wfc_p, bfc_p)
    return outp[:, :C]


# speedup vs baseline: 3.4545x; 3.4545x over previous
import functools

import jax
import jax.numpy as jnp
from jax import lax
from jax.experimental import pallas as pl
from jax.experimental.pallas import tpu as pltpu
from jax.experimental.pallas import tpu_sc as plsc

N = 10000
E = 320000
D = 128
H = 128
C = 2
G = 64
EPS = 1e-5

NP = 10240
NC = 2
NS = 16
L = 16
NW = NC * NS
EW = E // NW
K = 80
CH = EW // K
SLABA = NP // NS
SLAB = NP // NS
HIGHEST = jax.lax.Precision.HIGHEST

@functools.cache
def _mesh():
    return plsc.VectorSubcoreMesh(
        core_axis_name="c", subcore_axis_name="s",
        num_cores=NC, num_subcores=NS)


def _sc_deg_body(dst_hbm, out_hbm, didx_v, ones_v, zcol_v, acc_sh):
    core = lax.axis_index("c")
    sub = lax.axis_index("s")
    w = sub * NC + core

    for i in range(K // L):
        ones_v[pl.ds(i * L, L)] = jnp.ones((L,), jnp.float32)

    def _zero(i, _):
        zcol_v[pl.ds(i * L, L)] = jnp.zeros((L,), jnp.float32)
        return 0
    lax.fori_loop(0, SLAB // L, _zero, 0)
    pltpu.sync_copy(zcol_v, acc_sh.at[pl.ds(sub * SLAB, SLAB)])
    plsc.subcore_barrier()

    pltpu.sync_copy(dst_hbm.at[w], didx_v)

    def _chunk(j, _):
        pltpu.sync_copy(ones_v, acc_sh.at[didx_v.at[j]], add=True)
        return 0
    lax.fori_loop(0, CH, _chunk, 0)
    plsc.subcore_barrier()
    pltpu.sync_copy(acc_sh.at[pl.ds(sub * SLAB, SLAB)],
                    out_hbm.at[core, pl.ds(sub * SLAB, SLAB)])


@functools.cache
def _sc_deg():
    return pl.kernel(
        _sc_deg_body,
        out_type=jax.ShapeDtypeStruct((NC, NP), jnp.float32),
        mesh=_mesh(),
        scratch_types=[
            pltpu.VMEM((CH, K), jnp.int32),
            pltpu.VMEM((K,), jnp.float32),
            pltpu.VMEM((SLAB,), jnp.float32),
            pltpu.VMEM_SHARED((NP,), jnp.float32),
        ],
    )


def _sc_agg_body(y_hbm, src_hbm, dst_hbm, out_hbm, sidx0_v, sidx1_v, didx0_v,
                 didx1_v, rows0_v, rows1_v, isem0, isem1, gsem0, gsem1,
                 acc_sh):
    core = lax.axis_index("c")
    sub = lax.axis_index("s")
    w = sub * NC + core
    sidx = (sidx0_v, sidx1_v)
    didx = (didx0_v, didx1_v)
    rows = (rows0_v, rows1_v)
    isem = (isem0, isem1)
    gsem = (gsem0, gsem1)

    def _zero(i, _):
        rows0_v[i // 8, pl.ds((i % 8) * L, L)] = jnp.zeros((L,), jnp.float32)
        return 0
    lax.fori_loop(0, K * (H // L), _zero, 0)
    for t in range(SLABA // K):
        pltpu.sync_copy(rows0_v, acc_sh.at[pl.ds(sub * SLABA + t * K, K)])
    plsc.subcore_barrier()

    def _iload(j, b):
        pltpu.async_copy(src_hbm.at[w, j], sidx[b], isem[b])
        pltpu.async_copy(dst_hbm.at[w, j], didx[b], isem[b])

    def _iwait(j, b):
        pltpu.make_async_copy(src_hbm.at[w, j], sidx[b], isem[b]).wait()
        pltpu.make_async_copy(dst_hbm.at[w, j], didx[b], isem[b]).wait()

    def _gather(b):
        pltpu.async_copy(y_hbm.at[sidx[b]], rows[b], gsem[b])

    def _gwait(b):
        pltpu.make_async_copy(y_hbm.at[sidx[b]], rows[b], gsem[b]).wait()

    _iload(0, 0)
    _iload(1, 1)
    _iwait(0, 0)
    _gather(0)

    def _pair(i, _):
        for b in range(2):
            j = 2 * i + b
            nb = 1 - b
            _iwait(j + 1, nb)
            _gather(nb)
            _gwait(b)
            pltpu.sync_copy(rows[b], acc_sh.at[didx[b]], add=True)
            _iload(j + 2, b)
        return 0
    lax.fori_loop(0, (CH - 1) // 2, _pair, 0)
    _gwait(0)
    pltpu.sync_copy(rows0_v, acc_sh.at[didx0_v], add=True)
    _iwait(CH, 1)
    plsc.subcore_barrier()
    pltpu.sync_copy(acc_sh.at[pl.ds(sub * SLABA, SLABA)],
                    out_hbm.at[core, pl.ds(sub * SLABA, SLABA)])


@functools.cache
def _sc_agg():
    return pl.kernel(
        _sc_agg_body,
        out_type=jax.ShapeDtypeStruct((NC, NP, H), jnp.float32),
        mesh=_mesh(),
        scratch_types=[
            pltpu.VMEM((K,), jnp.int32),
            pltpu.VMEM((K,), jnp.int32),
            pltpu.VMEM((K,), jnp.int32),
            pltpu.VMEM((K,), jnp.int32),
            pltpu.VMEM((K, H), jnp.float32),
            pltpu.VMEM((K, H), jnp.float32),
            pltpu.SemaphoreType.DMA,
            pltpu.SemaphoreType.DMA,
            pltpu.SemaphoreType.DMA,
            pltpu.SemaphoreType.DMA,
            pltpu.VMEM_SHARED((NP, H), jnp.float32),
        ],
    )


def _tc_pre_body(x_ref, w1_ref, d0_ref, d1_ref, y1_ref, dinv_ref):
    deg = d0_ref[...] + d1_ref[...] + 1.0
    dinv = 1.0 / jnp.sqrt(deg)
    xw = jnp.dot(x_ref[...], w1_ref[...], precision=HIGHEST)
    y1_ref[...] = dinv * xw
    dinv_ref[...] = dinv


def _tc_stats_body(a0_ref, a1_ref, y_ref, dinv_ref, b_ref, batr_ref,
                   h_ref, scale_ref, shift_ref):
    h = dinv_ref[...] * (a0_ref[...] + a1_ref[...] + y_ref[...]) + b_ref[...]
    oht = (batr_ref[...] ==
           lax.broadcasted_iota(jnp.int32, (G, NP), 0)).astype(jnp.float32)
    rc = 1.0 / jnp.maximum(jnp.sum(oht, axis=1, keepdims=True), 1.0)
    mean = jnp.dot(oht, h, precision=HIGHEST) * rc
    ex2 = jnp.dot(oht, h * h, precision=HIGHEST) * rc
    var = ex2 - mean * mean
    scale = 1.0 / jnp.sqrt(var + EPS)
    h_ref[...] = h
    scale_ref[...] = scale
    shift_ref[...] = -mean * scale


def _tc_apply_body(h_ref, scale_ref, shift_ref, batc_ref, dinv_ref, w2_ref,
                   y2_ref):
    oh = (batc_ref[...] ==
          lax.broadcasted_iota(jnp.int32, (NP, G), 1)).astype(jnp.float32)
    hn = (h_ref[...] * jnp.dot(oh, scale_ref[...], precision=HIGHEST)
          + jnp.dot(oh, shift_ref[...], precision=HIGHEST))
    h1 = jnp.maximum(hn, 0.0)
    y2_ref[...] = dinv_ref[...] * jnp.dot(h1, w2_ref[...], precision=HIGHEST)


def _tc_pool_body(h_ref, scale_ref, shift_ref, batc_ref, batr_ref, wfc_ref,
                  bfc_ref, out_ref):
    oh = (batc_ref[...] ==
          lax.broadcasted_iota(jnp.int32, (NP, G), 1)).astype(jnp.float32)
    oht = (batr_ref[...] ==
           lax.broadcasted_iota(jnp.int32, (G, NP), 0)).astype(jnp.float32)
    rc = 1.0 / jnp.maximum(jnp.sum(oht, axis=1, keepdims=True), 1.0)
    hn = (h_ref[...] * jnp.dot(oh, scale_ref[...], precision=HIGHEST)
          + jnp.dot(oh, shift_ref[...], precision=HIGHEST))
    h2 = jnp.maximum(hn, 0.0)
    pooled = jnp.dot(oht, h2, precision=HIGHEST) * rc
    out_ref[...] = jnp.dot(pooled, wfc_ref[...], precision=HIGHEST) + bfc_ref[...]


_tc_pre = pl.pallas_call(
    _tc_pre_body,
    out_shape=[jax.ShapeDtypeStruct((NP, H), jnp.float32),
               jax.ShapeDtypeStruct((NP, 1), jnp.float32)])

_tc_stats = pl.pallas_call(
    _tc_stats_body,
    out_shape=[jax.ShapeDtypeStruct((NP, H), jnp.float32),
               jax.ShapeDtypeStruct((G, H), jnp.float32),
               jax.ShapeDtypeStruct((G, H), jnp.float32)])

_tc_apply = pl.pallas_call(
    _tc_apply_body,
    out_shape=jax.ShapeDtypeStruct((NP, H), jnp.float32))

_tc_pool = pl.pallas_call(
    _tc_pool_body,
    out_shape=jax.ShapeDtypeStruct((G, 128), jnp.float32))


def kernel(x, edge_index, batch, W1, b1, W2, b2, Wfc, bfc):
    dst3u = edge_index[1].reshape(NW, CH, K)
    src3 = jnp.pad(edge_index[0].reshape(NW, CH, K), ((0, 0), (0, 2), (0, 0)))
    dst3 = jnp.pad(dst3u, ((0, 0), (0, 2), (0, 0)))
    x_p = jnp.pad(x, ((0, NP - N), (0, 0)))
    batc = jnp.pad(batch.astype(jnp.int32), (0, NP - N),
                   constant_values=G).reshape(NP, 1)
    batr = batc.reshape(1, NP)
    wfc_p = jnp.pad(Wfc, ((0, 0), (0, 128 - C)))
    bfc_p = jnp.pad(bfc, (0, 128 - C)).reshape(1, 128)

    degp = _sc_deg()(dst3u)
    d0 = degp[0].reshape(NP, 1)
    d1 = degp[1].reshape(NP, 1)
    y1, dinv = _tc_pre(x_p, W1, d0, d1)

    ag1 = _sc_agg()(y1, src3, dst3)
    h1, sc1, sh1 = _tc_stats(ag1[0], ag1[1], y1, dinv, b1.reshape(1, H), batr)
    y2 = _tc_apply(h1, sc1, sh1, batc, dinv, W2)

    ag2 = _sc_agg()(y2, src3, dst3)
    h2, sc2, sh2 = _tc_stats(ag2[0], ag2[1], y2, dinv, b2.reshape(1, H), batr)
    outp = _tc_pool(h2, sc2, sh2, batc, batr, wfc_p, bfc_p)
    return outp[:, :C]


# speedup vs baseline: 3.7940x; 1.0983x over previous
import functools

import jax
import jax.numpy as jnp
from jax import lax
from jax.experimental import pallas as pl
from jax.experimental.pallas import tpu as pltpu
from jax.experimental.pallas import tpu_sc as plsc

N = 10000
E = 320000
D = 128
H = 128
C = 2
G = 64
EPS = 1e-5

NP = 10240
NC = 2
NS = 16
L = 16
NW = NC * NS
EW = E // NW
KA = 125
CHA = 80
KD = 80
CHD = 125
KZ = 40
SLABA = NP // NS
SLAB = NP // NS
HIGHEST = jax.lax.Precision.HIGHEST

@functools.cache
def _mesh():
    return plsc.VectorSubcoreMesh(
        core_axis_name="c", subcore_axis_name="s",
        num_cores=NC, num_subcores=NS)


def _sc_deg_body(dst_hbm, out_hbm, didx_v, ones_v, zcol_v, acc_sh):
    core = lax.axis_index("c")
    sub = lax.axis_index("s")
    w = sub * NC + core

    for i in range(KD // L):
        ones_v[pl.ds(i * L, L)] = jnp.ones((L,), jnp.float32)

    def _zero(i, _):
        zcol_v[pl.ds(i * L, L)] = jnp.zeros((L,), jnp.float32)
        return 0
    lax.fori_loop(0, SLAB // L, _zero, 0)
    pltpu.sync_copy(zcol_v, acc_sh.at[pl.ds(sub * SLAB, SLAB)])
    plsc.subcore_barrier()

    pltpu.sync_copy(dst_hbm.at[w], didx_v)

    def _chunk(j, _):
        pltpu.sync_copy(ones_v, acc_sh.at[didx_v.at[j]], add=True)
        return 0
    lax.fori_loop(0, CHD, _chunk, 0)
    plsc.subcore_barrier()
    pltpu.sync_copy(acc_sh.at[pl.ds(sub * SLAB, SLAB)],
                    out_hbm.at[core, pl.ds(sub * SLAB, SLAB)])


@functools.cache
def _sc_deg():
    return pl.kernel(
        _sc_deg_body,
        out_type=jax.ShapeDtypeStruct((NC, NP), jnp.float32),
        mesh=_mesh(),
        scratch_types=[
            pltpu.VMEM((CHD, KD), jnp.int32),
            pltpu.VMEM((KD,), jnp.float32),
            pltpu.VMEM((SLAB,), jnp.float32),
            pltpu.VMEM_SHARED((NP,), jnp.float32),
        ],
    )


def _sc_agg_body(y_hbm, src_hbm, dst_hbm, out_hbm, sidx0_v, sidx1_v, didx0_v,
                 didx1_v, rows0_v, rows1_v, isem0, isem1, gsem0, gsem1,
                 acc_sh):
    core = lax.axis_index("c")
    sub = lax.axis_index("s")
    w = sub * NC + core
    sidx = (sidx0_v, sidx1_v)
    didx = (didx0_v, didx1_v)
    rows = (rows0_v, rows1_v)
    isem = (isem0, isem1)
    gsem = (gsem0, gsem1)

    def _zero(i, _):
        rows0_v[i // 8, pl.ds((i % 8) * L, L)] = jnp.zeros((L,), jnp.float32)
        return 0
    lax.fori_loop(0, KA * (H // L), _zero, 0)
    for t in range(SLABA // KZ):
        pltpu.sync_copy(rows0_v.at[pl.ds(0, KZ)],
                        acc_sh.at[pl.ds(sub * SLABA + t * KZ, KZ)])
    plsc.subcore_barrier()

    def _iload(j, b):
        pltpu.async_copy(src_hbm.at[w, j], sidx[b], isem[b])
        pltpu.async_copy(dst_hbm.at[w, j], didx[b], isem[b])

    def _iwait(j, b):
        pltpu.make_async_copy(src_hbm.at[w, j], sidx[b], isem[b]).wait()
        pltpu.make_async_copy(dst_hbm.at[w, j], didx[b], isem[b]).wait()

    def _gather(b):
        pltpu.async_copy(y_hbm.at[sidx[b]], rows[b], gsem[b])

    def _gwait(b):
        pltpu.make_async_copy(y_hbm.at[sidx[b]], rows[b], gsem[b]).wait()

    _iload(0, 0)
    _iload(1, 1)
    _iwait(0, 0)
    _gather(0)

    def _pair(i, _):
        for b in range(2):
            j = 2 * i + b
            nb = 1 - b
            _iwait(j + 1, nb)
            _gather(nb)
            _gwait(b)
            pltpu.sync_copy(rows[b], acc_sh.at[didx[b]], add=True)
            _iload(j + 2, b)
        return 0
    lax.fori_loop(0, (CHA - 2) // 2, _pair, 0)
    _iwait(CHA - 1, 1)
    _gather(1)
    _gwait(0)
    pltpu.sync_copy(rows0_v, acc_sh.at[didx0_v], add=True)
    _gwait(1)
    pltpu.sync_copy(rows1_v, acc_sh.at[didx1_v], add=True)
    plsc.subcore_barrier()
    pltpu.sync_copy(acc_sh.at[pl.ds(sub * SLABA, SLABA)],
                    out_hbm.at[core, pl.ds(sub * SLABA, SLABA)])


@functools.cache
def _sc_agg():
    return pl.kernel(
        _sc_agg_body,
        out_type=jax.ShapeDtypeStruct((NC, NP, H), jnp.float32),
        mesh=_mesh(),
        scratch_types=[
            pltpu.VMEM((KA,), jnp.int32),
            pltpu.VMEM((KA,), jnp.int32),
            pltpu.VMEM((KA,), jnp.int32),
            pltpu.VMEM((KA,), jnp.int32),
            pltpu.VMEM((KA, H), jnp.float32),
            pltpu.VMEM((KA, H), jnp.float32),
            pltpu.SemaphoreType.DMA,
            pltpu.SemaphoreType.DMA,
            pltpu.SemaphoreType.DMA,
            pltpu.SemaphoreType.DMA,
            pltpu.VMEM_SHARED((NP, H), jnp.float32),
        ],
    )


def _tc_pre_body(x_ref, w1_ref, d0_ref, d1_ref, y1_ref, dinv_ref):
    deg = d0_ref[...] + d1_ref[...] + 1.0
    dinv = 1.0 / jnp.sqrt(deg)
    xw = jnp.dot(x_ref[...], w1_ref[...], precision=HIGHEST)
    y1_ref[...] = dinv * xw
    dinv_ref[...] = dinv


def _tc_stats_body(a0_ref, a1_ref, y_ref, dinv_ref, b_ref, batr_ref,
                   h_ref, scale_ref, shift_ref):
    h = dinv_ref[...] * (a0_ref[...] + a1_ref[...] + y_ref[...]) + b_ref[...]
    oht = (batr_ref[...] ==
           lax.broadcasted_iota(jnp.int32, (G, NP), 0)).astype(jnp.float32)
    rc = 1.0 / jnp.maximum(jnp.sum(oht, axis=1, keepdims=True), 1.0)
    mean = jnp.dot(oht, h, precision=HIGHEST) * rc
    ex2 = jnp.dot(oht, h * h, precision=HIGHEST) * rc
    var = ex2 - mean * mean
    scale = 1.0 / jnp.sqrt(var + EPS)
    h_ref[...] = h
    scale_ref[...] = scale
    shift_ref[...] = -mean * scale


def _tc_apply_body(h_ref, scale_ref, shift_ref, batc_ref, dinv_ref, w2_ref,
                   y2_ref):
    oh = (batc_ref[...] ==
          lax.broadcasted_iota(jnp.int32, (NP, G), 1)).astype(jnp.float32)
    hn = (h_ref[...] * jnp.dot(oh, scale_ref[...], precision=HIGHEST)
          + jnp.dot(oh, shift_ref[...], precision=HIGHEST))
    h1 = jnp.maximum(hn, 0.0)
    y2_ref[...] = dinv_ref[...] * jnp.dot(h1, w2_ref[...], precision=HIGHEST)


def _tc_pool_body(h_ref, scale_ref, shift_ref, batc_ref, batr_ref, wfc_ref,
                  bfc_ref, out_ref):
    oh = (batc_ref[...] ==
          lax.broadcasted_iota(jnp.int32, (NP, G), 1)).astype(jnp.float32)
    oht = (batr_ref[...] ==
           lax.broadcasted_iota(jnp.int32, (G, NP), 0)).astype(jnp.float32)
    rc = 1.0 / jnp.maximum(jnp.sum(oht, axis=1, keepdims=True), 1.0)
    hn = (h_ref[...] * jnp.dot(oh, scale_ref[...], precision=HIGHEST)
          + jnp.dot(oh, shift_ref[...], precision=HIGHEST))
    h2 = jnp.maximum(hn, 0.0)
    pooled = jnp.dot(oht, h2, precision=HIGHEST) * rc
    out_ref[...] = jnp.dot(pooled, wfc_ref[...], precision=HIGHEST) + bfc_ref[...]


_tc_pre = pl.pallas_call(
    _tc_pre_body,
    out_shape=[jax.ShapeDtypeStruct((NP, H), jnp.float32),
               jax.ShapeDtypeStruct((NP, 1), jnp.float32)])

_tc_stats = pl.pallas_call(
    _tc_stats_body,
    out_shape=[jax.ShapeDtypeStruct((NP, H), jnp.float32),
               jax.ShapeDtypeStruct((G, H), jnp.float32),
               jax.ShapeDtypeStruct((G, H), jnp.float32)])

_tc_apply = pl.pallas_call(
    _tc_apply_body,
    out_shape=jax.ShapeDtypeStruct((NP, H), jnp.float32))

_tc_pool = pl.pallas_call(
    _tc_pool_body,
    out_shape=jax.ShapeDtypeStruct((G, 128), jnp.float32))


def kernel(x, edge_index, batch, W1, b1, W2, b2, Wfc, bfc):
    dst3u = edge_index[1].reshape(NW, CHD, KD)
    src3 = edge_index[0].reshape(NW, CHA, KA)
    dst3 = edge_index[1].reshape(NW, CHA, KA)
    x_p = jnp.pad(x, ((0, NP - N), (0, 0)))
    batc = jnp.pad(batch.astype(jnp.int32), (0, NP - N),
                   constant_values=G).reshape(NP, 1)
    batr = batc.reshape(1, NP)
    wfc_p = jnp.pad(Wfc, ((0, 0), (0, 128 - C)))
    bfc_p = jnp.pad(bfc, (0, 128 - C)).reshape(1, 128)

    degp = _sc_deg()(dst3u)
    d0 = degp[0].reshape(NP, 1)
    d1 = degp[1].reshape(NP, 1)
    y1, dinv = _tc_pre(x_p, W1, d0, d1)

    ag1 = _sc_agg()(y1, src3, dst3)
    h1, sc1, sh1 = _tc_stats(ag1[0], ag1[1], y1, dinv, b1.reshape(1, H), batr)
    y2 = _tc_apply(h1, sc1, sh1, batc, dinv, W2)

    ag2 = _sc_agg()(y2, src3, dst3)
    h2, sc2, sh2 = _tc_stats(ag2[0], ag2[1], y2, dinv, b2.reshape(1, H), batr)
    outp = _tc_pool(h2, sc2, sh2, batc, batr, wfc_p, bfc_p)
    return outp[:, :C]


# speedup vs baseline: 3.8244x; 1.0080x over previous
import functools

import jax
import jax.numpy as jnp
from jax import lax
from jax.experimental import pallas as pl
from jax.experimental.pallas import tpu as pltpu
from jax.experimental.pallas import tpu_sc as plsc

N = 10000
E = 320000
D = 128
H = 128
C = 2
G = 64
EPS = 1e-5

NP = 10240
NC = 2
NS = 16
L = 16
NW = NC * NS
EW = E // NW
KA = 125
CHA = 80
KD = 80
CHD = 125
KZ = 40
SLABA = NP // NS
SLAB = NP // NS
HIGHEST = jax.lax.Precision.HIGHEST

@functools.cache
def _mesh():
    return plsc.VectorSubcoreMesh(
        core_axis_name="c", subcore_axis_name="s",
        num_cores=NC, num_subcores=NS)


def _sc_deg_body(dst_hbm, out_hbm, didx_v, ones_v, zcol_v, acc_sh):
    core = lax.axis_index("c")
    sub = lax.axis_index("s")
    w = sub * NC + core

    for i in range(KD // L):
        ones_v[pl.ds(i * L, L)] = jnp.ones((L,), jnp.float32)

    def _zero(i, _):
        zcol_v[pl.ds(i * L, L)] = jnp.zeros((L,), jnp.float32)
        return 0
    lax.fori_loop(0, SLAB // L, _zero, 0)
    pltpu.sync_copy(zcol_v, acc_sh.at[pl.ds(sub * SLAB, SLAB)])
    plsc.subcore_barrier()

    pltpu.sync_copy(dst_hbm.at[w], didx_v)

    def _chunk(j, _):
        pltpu.sync_copy(ones_v, acc_sh.at[didx_v.at[j]], add=True)
        return 0
    lax.fori_loop(0, CHD, _chunk, 0)
    plsc.subcore_barrier()
    pltpu.sync_copy(acc_sh.at[pl.ds(sub * SLAB, SLAB)],
                    out_hbm.at[core, pl.ds(sub * SLAB, SLAB)])


@functools.cache
def _sc_deg():
    return pl.kernel(
        _sc_deg_body,
        out_type=jax.ShapeDtypeStruct((NC, NP), jnp.float32),
        mesh=_mesh(),
        scratch_types=[
            pltpu.VMEM((CHD, KD), jnp.int32),
            pltpu.VMEM((KD,), jnp.float32),
            pltpu.VMEM((SLAB,), jnp.float32),
            pltpu.VMEM_SHARED((NP,), jnp.float32),
        ],
    )


def _sc_agg_body(y_hbm, src_hbm, dst_hbm, out_hbm, sidx0_v, sidx1_v, didx0_v,
                 didx1_v, rows0_v, rows1_v, isem0, isem1, gsem0, gsem1,
                 acc_sh):
    core = lax.axis_index("c")
    sub = lax.axis_index("s")
    w = sub * NC + core
    sidx = (sidx0_v, sidx1_v)
    didx = (didx0_v, didx1_v)
    rows = (rows0_v, rows1_v)
    isem = (isem0, isem1)
    gsem = (gsem0, gsem1)

    def _zero(i, _):
        rows0_v[i // 8, pl.ds((i % 8) * L, L)] = jnp.zeros((L,), jnp.float32)
        return 0
    lax.fori_loop(0, KA * (H // L), _zero, 0)
    for t in range(SLABA // KZ):
        pltpu.sync_copy(rows0_v.at[pl.ds(0, KZ)],
                        acc_sh.at[pl.ds(sub * SLABA + t * KZ, KZ)])
    plsc.subcore_barrier()

    def _iload(j, b):
        pltpu.async_copy(src_hbm.at[w, j], sidx[b], isem[b])
        pltpu.async_copy(dst_hbm.at[w, j], didx[b], isem[b])

    def _iwait(j, b):
        pltpu.make_async_copy(src_hbm.at[w, j], sidx[b], isem[b]).wait()
        pltpu.make_async_copy(dst_hbm.at[w, j], didx[b], isem[b]).wait()

    def _gather(b):
        pltpu.async_copy(y_hbm.at[sidx[b]], rows[b], gsem[b])

    def _gwait(b):
        pltpu.make_async_copy(y_hbm.at[sidx[b]], rows[b], gsem[b]).wait()

    _iload(0, 0)
    _iload(1, 1)
    _iwait(0, 0)
    _gather(0)

    def _pair(i, _):
        for b in range(2):
            j = 2 * i + b
            nb = 1 - b
            _iwait(j + 1, nb)
            _gather(nb)
            _gwait(b)
            pltpu.sync_copy(rows[b], acc_sh.at[didx[b]], add=True)
            _iload(j + 2, b)
        return 0
    lax.fori_loop(0, (CHA - 2) // 2, _pair, 0)
    _iwait(CHA - 1, 1)
    _gather(1)
    _gwait(0)
    pltpu.sync_copy(rows0_v, acc_sh.at[didx0_v], add=True)
    _gwait(1)
    pltpu.sync_copy(rows1_v, acc_sh.at[didx1_v], add=True)
    plsc.subcore_barrier()
    pltpu.sync_copy(acc_sh.at[pl.ds(sub * SLABA, SLABA)],
                    out_hbm.at[core, pl.ds(sub * SLABA, SLABA)])


@functools.cache
def _sc_agg():
    return pl.kernel(
        _sc_agg_body,
        out_type=jax.ShapeDtypeStruct((NC, NP, H), jnp.float32),
        mesh=_mesh(),
        scratch_types=[
            pltpu.VMEM((KA,), jnp.int32),
            pltpu.VMEM((KA,), jnp.int32),
            pltpu.VMEM((KA,), jnp.int32),
            pltpu.VMEM((KA,), jnp.int32),
            pltpu.VMEM((KA, H), jnp.float32),
            pltpu.VMEM((KA, H), jnp.float32),
            pltpu.SemaphoreType.DMA,
            pltpu.SemaphoreType.DMA,
            pltpu.SemaphoreType.DMA,
            pltpu.SemaphoreType.DMA,
            pltpu.VMEM_SHARED((NP, H), jnp.float32),
        ],
    )


def _tc_mm_body(x_ref, w1_ref, xw_ref):
    xw_ref[...] = jnp.dot(x_ref[...], w1_ref[...], precision=HIGHEST)


def _tc_scale_body(xw_ref, d0_ref, d1_ref, y1_ref, dinv_ref):
    deg = d0_ref[...] + d1_ref[...] + 1.0
    dinv = 1.0 / jnp.sqrt(deg)
    y1_ref[...] = dinv * xw_ref[...]
    dinv_ref[...] = dinv


def _tc_stats_body(a0_ref, a1_ref, y_ref, dinv_ref, b_ref, batr_ref,
                   h_ref, scale_ref, shift_ref):
    h = dinv_ref[...] * (a0_ref[...] + a1_ref[...] + y_ref[...]) + b_ref[...]
    oht = (batr_ref[...] ==
           lax.broadcasted_iota(jnp.int32, (G, NP), 0)).astype(jnp.float32)
    rc = 1.0 / jnp.maximum(jnp.sum(oht, axis=1, keepdims=True), 1.0)
    mean = jnp.dot(oht, h, precision=HIGHEST) * rc
    ex2 = jnp.dot(oht, h * h, precision=HIGHEST) * rc
    var = ex2 - mean * mean
    scale = 1.0 / jnp.sqrt(var + EPS)
    h_ref[...] = h
    scale_ref[...] = scale
    shift_ref[...] = -mean * scale


def _tc_apply_body(h_ref, scale_ref, shift_ref, batc_ref, dinv_ref, w2_ref,
                   y2_ref):
    oh = (batc_ref[...] ==
          lax.broadcasted_iota(jnp.int32, (NP, G), 1)).astype(jnp.float32)
    hn = (h_ref[...] * jnp.dot(oh, scale_ref[...], precision=HIGHEST)
          + jnp.dot(oh, shift_ref[...], precision=HIGHEST))
    h1 = jnp.maximum(hn, 0.0)
    y2_ref[...] = dinv_ref[...] * jnp.dot(h1, w2_ref[...], precision=HIGHEST)


def _tc_pool_body(h_ref, scale_ref, shift_ref, batc_ref, batr_ref, wfc_ref,
                  bfc_ref, out_ref):
    oh = (batc_ref[...] ==
          lax.broadcasted_iota(jnp.int32, (NP, G), 1)).astype(jnp.float32)
    oht = (batr_ref[...] ==
           lax.broadcasted_iota(jnp.int32, (G, NP), 0)).astype(jnp.float32)
    rc = 1.0 / jnp.maximum(jnp.sum(oht, axis=1, keepdims=True), 1.0)
    hn = (h_ref[...] * jnp.dot(oh, scale_ref[...], precision=HIGHEST)
          + jnp.dot(oh, shift_ref[...], precision=HIGHEST))
    h2 = jnp.maximum(hn, 0.0)
    pooled = jnp.dot(oht, h2, precision=HIGHEST) * rc
    out_ref[...] = jnp.dot(pooled, wfc_ref[...], precision=HIGHEST) + bfc_ref[...]


_tc_mm = pl.pallas_call(
    _tc_mm_body,
    out_shape=jax.ShapeDtypeStruct((NP, H), jnp.float32))

_tc_scale = pl.pallas_call(
    _tc_scale_body,
    out_shape=[jax.ShapeDtypeStruct((NP, H), jnp.float32),
               jax.ShapeDtypeStruct((NP, 1), jnp.float32)])

_tc_stats = pl.pallas_call(
    _tc_stats_body,
    out_shape=[jax.ShapeDtypeStruct((NP, H), jnp.float32),
               jax.ShapeDtypeStruct((G, H), jnp.float32),
               jax.ShapeDtypeStruct((G, H), jnp.float32)])

_tc_apply = pl.pallas_call(
    _tc_apply_body,
    out_shape=jax.ShapeDtypeStruct((NP, H), jnp.float32))

_tc_pool = pl.pallas_call(
    _tc_pool_body,
    out_shape=jax.ShapeDtypeStruct((G, 128), jnp.float32))


def kernel(x, edge_index, batch, W1, b1, W2, b2, Wfc, bfc):
    dst3u = edge_index[1].reshape(NW, CHD, KD)
    src3 = edge_index[0].reshape(NW, CHA, KA)
    dst3 = edge_index[1].reshape(NW, CHA, KA)
    x_p = jnp.pad(x, ((0, NP - N), (0, 0)))
    batc = jnp.pad(batch.astype(jnp.int32), (0, NP - N),
                   constant_values=G).reshape(NP, 1)
    batr = batc.reshape(1, NP)
    wfc_p = jnp.pad(Wfc, ((0, 0), (0, 128 - C)))
    bfc_p = jnp.pad(bfc, (0, 128 - C)).reshape(1, 128)

    degp = _sc_deg()(dst3u)
    xw1 = _tc_mm(x_p, W1)
    d0 = degp[0].reshape(NP, 1)
    d1 = degp[1].reshape(NP, 1)
    y1, dinv = _tc_scale(xw1, d0, d1)

    ag1 = _sc_agg()(y1, src3, dst3)
    h1, sc1, sh1 = _tc_stats(ag1[0], ag1[1], y1, dinv, b1.reshape(1, H), batr)
    y2 = _tc_apply(h1, sc1, sh1, batc, dinv, W2)

    ag2 = _sc_agg()(y2, src3, dst3)
    h2, sc2, sh2 = _tc_stats(ag2[0], ag2[1], y2, dinv, b2.reshape(1, H), batr)
    outp = _tc_pool(h2, sc2, sh2, batc, batr, wfc_p, bfc_p)
    return outp[:, :C]


# speedup vs baseline: 5.8227x; 1.5225x over previous
import functools

import jax
import jax.numpy as jnp
from jax import lax
from jax.experimental import pallas as pl
from jax.experimental.pallas import tpu as pltpu
from jax.experimental.pallas import tpu_sc as plsc

N = 10000
E = 320000
D = 128
H = 128
C = 2
G = 64
EPS = 1e-5

NP = 10240
NC = 2
NS = 16
L = 16
NW = NC * NS
EW = E // NW
KA = 125
CHA = 80
KD = 80
CHD = 125
KZ = 40
SLABA = NP // NS
SLAB = NP // NS
HIGHEST = jax.lax.Precision.HIGHEST

@functools.cache
def _mesh():
    return plsc.VectorSubcoreMesh(
        core_axis_name="c", subcore_axis_name="s",
        num_cores=NC, num_subcores=NS)


def _sc_deg_body(dst_hbm, out_hbm, didx_v, ones_v, zcol_v, acc_sh):
    core = lax.axis_index("c")
    sub = lax.axis_index("s")
    w = sub * NC + core

    for i in range(KD // L):
        ones_v[pl.ds(i * L, L)] = jnp.ones((L,), jnp.float32)

    def _zero(i, _):
        zcol_v[pl.ds(i * L, L)] = jnp.zeros((L,), jnp.float32)
        return 0
    lax.fori_loop(0, SLAB // L, _zero, 0)
    pltpu.sync_copy(zcol_v, acc_sh.at[pl.ds(sub * SLAB, SLAB)])
    plsc.subcore_barrier()

    pltpu.sync_copy(dst_hbm.at[w], didx_v)

    def _chunk(j, _):
        pltpu.sync_copy(ones_v, acc_sh.at[didx_v.at[j]], add=True)
        return 0
    lax.fori_loop(0, CHD, _chunk, 0)
    plsc.subcore_barrier()
    pltpu.sync_copy(acc_sh.at[pl.ds(sub * SLAB, SLAB)],
                    out_hbm.at[core, pl.ds(sub * SLAB, SLAB)])


@functools.cache
def _sc_deg():
    return pl.kernel(
        _sc_deg_body,
        out_type=jax.ShapeDtypeStruct((NC, NP), jnp.float32),
        mesh=_mesh(),
        scratch_types=[
            pltpu.VMEM((CHD, KD), jnp.int32),
            pltpu.VMEM((KD,), jnp.float32),
            pltpu.VMEM((SLAB,), jnp.float32),
            pltpu.VMEM_SHARED((NP,), jnp.float32),
        ],
    )


def _sc_agg_body(y_hbm, src_hbm, dst_hbm, out_hbm, sidx0_v, sidx1_v, didx0_v,
                 didx1_v, rows0_v, rows1_v, isem0, isem1, gsem0, gsem1,
                 acc_sh):
    core = lax.axis_index("c")
    sub = lax.axis_index("s")
    w = sub * NC + core
    sidx = (sidx0_v, sidx1_v)
    didx = (didx0_v, didx1_v)
    rows = (rows0_v, rows1_v)
    isem = (isem0, isem1)
    gsem = (gsem0, gsem1)

    def _zero(i, _):
        rows0_v[i // 8, pl.ds((i % 8) * L, L)] = jnp.zeros((L,), jnp.float32)
        return 0
    lax.fori_loop(0, KA * (H // L), _zero, 0)
    for t in range(SLABA // KZ):
        pltpu.sync_copy(rows0_v.at[pl.ds(0, KZ)],
                        acc_sh.at[pl.ds(sub * SLABA + t * KZ, KZ)])
    plsc.subcore_barrier()

    def _iload(j, b):
        pltpu.async_copy(src_hbm.at[w, j], sidx[b], isem[b])
        pltpu.async_copy(dst_hbm.at[w, j], didx[b], isem[b])

    def _iwait(j, b):
        pltpu.make_async_copy(src_hbm.at[w, j], sidx[b], isem[b]).wait()
        pltpu.make_async_copy(dst_hbm.at[w, j], didx[b], isem[b]).wait()

    def _gather(b):
        pltpu.async_copy(y_hbm.at[sidx[b]], rows[b], gsem[b])

    def _gwait(b):
        pltpu.make_async_copy(y_hbm.at[sidx[b]], rows[b], gsem[b]).wait()

    _iload(0, 0)
    _iload(1, 1)
    _iwait(0, 0)
    _gather(0)

    def _pair(i, _):
        for b in range(2):
            j = 2 * i + b
            nb = 1 - b
            _iwait(j + 1, nb)
            _gather(nb)
            _gwait(b)
            pltpu.sync_copy(rows[b], acc_sh.at[didx[b]], add=True)
            _iload(j + 2, b)
        return 0
    lax.fori_loop(0, (CHA - 2) // 2, _pair, 0)
    _iwait(CHA - 1, 1)
    _gather(1)
    _gwait(0)
    pltpu.sync_copy(rows0_v, acc_sh.at[didx0_v], add=True)
    _gwait(1)
    pltpu.sync_copy(rows1_v, acc_sh.at[didx1_v], add=True)
    plsc.subcore_barrier()
    pltpu.sync_copy(acc_sh.at[pl.ds(sub * SLABA, SLABA)],
                    out_hbm.at[core, pl.ds(sub * SLABA, SLABA)])


@functools.cache
def _sc_agg():
    return pl.kernel(
        _sc_agg_body,
        out_type=jax.ShapeDtypeStruct((NC, NP, H), jnp.float32),
        mesh=_mesh(),
        scratch_types=[
            pltpu.VMEM((KA,), jnp.int32),
            pltpu.VMEM((KA,), jnp.int32),
            pltpu.VMEM((KA,), jnp.int32),
            pltpu.VMEM((KA,), jnp.int32),
            pltpu.VMEM((KA, H), jnp.float32),
            pltpu.VMEM((KA, H), jnp.float32),
            pltpu.SemaphoreType.DMA,
            pltpu.SemaphoreType.DMA,
            pltpu.SemaphoreType.DMA,
            pltpu.SemaphoreType.DMA,
            pltpu.VMEM_SHARED((NP, H), jnp.float32),
        ],
    )


def _tc_mm_body(x_ref, w1_ref, xw_ref):
    xw_ref[...] = jnp.dot(x_ref[...], w1_ref[...], precision=HIGHEST)


def _tc_scale_body(xw_ref, d0_ref, d1_ref, y1_ref, dinv_ref):
    deg = d0_ref[...] + d1_ref[...] + 1.0
    dinv = 1.0 / jnp.sqrt(deg)
    y1_ref[...] = dinv * xw_ref[...]
    dinv_ref[...] = dinv


def _norm_apply(a0, a1, y, dinv, b, batc, batr):
    h = dinv * (a0 + a1 + y) + b
    oht = (batr ==
           lax.broadcasted_iota(jnp.int32, (G, NP), 0)).astype(jnp.float32)
    rc = 1.0 / jnp.maximum(jnp.sum(oht, axis=1, keepdims=True), 1.0)
    mean = jnp.dot(oht, h, precision=HIGHEST) * rc
    ex2 = jnp.dot(oht, h * h, precision=HIGHEST) * rc
    var = ex2 - mean * mean
    scale = 1.0 / jnp.sqrt(var + EPS)
    shift = -mean * scale
    oh = (batc ==
          lax.broadcasted_iota(jnp.int32, (NP, G), 1)).astype(jnp.float32)
    hn = h * jnp.dot(oh, scale) + jnp.dot(oh, shift)
    return jnp.maximum(hn, 0.0), oht, rc


def _tc_stats_body(a0_ref, a1_ref, y_ref, dinv_ref, b_ref, batr_ref,
                   h_ref, scale_ref, shift_ref):
    h = dinv_ref[...] * (a0_ref[...] + a1_ref[...] + y_ref[...]) + b_ref[...]
    oht = (batr_ref[...] ==
           lax.broadcasted_iota(jnp.int32, (G, NP), 0)).astype(jnp.float32)
    rc = 1.0 / jnp.maximum(jnp.sum(oht, axis=1, keepdims=True), 1.0)
    mean = jnp.dot(oht, h, precision=HIGHEST) * rc
    ex2 = jnp.dot(oht, h * h, precision=HIGHEST) * rc
    var = ex2 - mean * mean
    scale = 1.0 / jnp.sqrt(var + EPS)
    h_ref[...] = h
    scale_ref[...] = scale
    shift_ref[...] = -mean * scale


def _tc_apply_body(h_ref, scale_ref, shift_ref, batc_ref, dinv_ref, w2_ref,
                   y2_ref):
    oh = (batc_ref[...] ==
          lax.broadcasted_iota(jnp.int32, (NP, G), 1)).astype(jnp.float32)
    hn = (h_ref[...] * jnp.dot(oh, scale_ref[...], precision=HIGHEST)
          + jnp.dot(oh, shift_ref[...], precision=HIGHEST))
    h1 = jnp.maximum(hn, 0.0)
    y2_ref[...] = dinv_ref[...] * jnp.dot(h1, w2_ref[...], precision=HIGHEST)


def _tc_layer2_body(a0_ref, a1_ref, y_ref, dinv_ref, b_ref, batc_ref,
                    batr_ref, wfc_ref, bfc_ref, out_ref):
    h2, oht, rc = _norm_apply(a0_ref[...], a1_ref[...], y_ref[...],
                              dinv_ref[...], b_ref[...], batc_ref[...],
                              batr_ref[...])
    pooled = jnp.dot(oht, h2, precision=HIGHEST) * rc
    out_ref[...] = jnp.dot(pooled, wfc_ref[...], precision=HIGHEST) + bfc_ref[...]


_tc_mm = pl.pallas_call(
    _tc_mm_body,
    out_shape=jax.ShapeDtypeStruct((NP, H), jnp.float32))

_tc_scale = pl.pallas_call(
    _tc_scale_body,
    out_shape=[jax.ShapeDtypeStruct((NP, H), jnp.float32),
               jax.ShapeDtypeStruct((NP, 1), jnp.float32)])

_tc_stats = pl.pallas_call(
    _tc_stats_body,
    out_shape=[jax.ShapeDtypeStruct((NP, H), jnp.float32),
               jax.ShapeDtypeStruct((G, H), jnp.float32),
               jax.ShapeDtypeStruct((G, H), jnp.float32)])

_tc_apply = pl.pallas_call(
    _tc_apply_body,
    out_shape=jax.ShapeDtypeStruct((NP, H), jnp.float32))

_tc_layer2 = pl.pallas_call(
    _tc_layer2_body,
    out_shape=jax.ShapeDtypeStruct((G, 128), jnp.float32))


def kernel(x, edge_index, batch, W1, b1, W2, b2, Wfc, bfc):
    dst3u = edge_index[1].reshape(NW, CHD, KD)
    src3 = edge_index[0].reshape(NW, CHA, KA)
    dst3 = edge_index[1].reshape(NW, CHA, KA)
    x_p = jnp.pad(x, ((0, NP - N), (0, 0)))
    batc = jnp.pad(batch.astype(jnp.int32), (0, NP - N),
                   constant_values=G).reshape(NP, 1)
    batr = batc.reshape(1, NP)
    wfc_p = jnp.pad(Wfc, ((0, 0), (0, 128 - C)))
    bfc_p = jnp.pad(bfc, (0, 128 - C)).reshape(1, 128)

    degp = _sc_deg()(dst3u)
    xw1 = _tc_mm(x_p, W1)
    d0 = degp[0].reshape(NP, 1)
    d1 = degp[1].reshape(NP, 1)
    y1, dinv = _tc_scale(xw1, d0, d1)

    ag1 = _sc_agg()(y1, src3, dst3)
    h1, sc1, sh1 = _tc_stats(ag1[0], ag1[1], y1, dinv, b1.reshape(1, H), batr)
    y2 = _tc_apply(h1, sc1, sh1, batc, dinv, W2)

    ag2 = _sc_agg()(y2, src3, dst3)
    outp = _tc_layer2(ag2[0], ag2[1], y2, dinv, b2.reshape(1, H), batc, batr,
                      wfc_p, bfc_p)
    return outp[:, :C]
